# baseline (speedup 1.0000x reference)
import functools

import jax
import jax.numpy as jnp
from jax import lax
from jax.experimental import pallas as pl
from jax.experimental.pallas import tpu as pltpu
from jax.experimental.pallas import tpu_sc as plsc

_B = 16384
_NCONT = 13
_D = 32
_F = 26
_NR = _NCONT + _F
_V = 100001

_NW = 32
_BPW = _B // _NW
_NB = 64
_NCHUNK = _BPW // _NB


def _make_sc_kernel():
    mesh = plsc.VectorSubcoreMesh(core_axis_name="c", subcore_axis_name="s")

    @functools.partial(
        pl.kernel,
        mesh=mesh,
        out_type=jax.ShapeDtypeStruct((_B * _NR, _D), jnp.float32),
        compiler_params=pltpu.CompilerParams(use_tc_tiling_on_sc=False),
        scratch_types=[
            pltpu.VMEM((_NB, _F), jnp.int32),
            pltpu.VMEM((_NB * _NR, _D), jnp.float32),
            pltpu.VMEM((_NB * _NCONT + 16,), jnp.float32),
            pltpu.VMEM((_NCONT, _D), jnp.float32),
            pltpu.VMEM((_NCONT, _D), jnp.float32),
            pltpu.SemaphoreType.DMA,
        ],
    )
    def sc_kernel(gid_hbm, x_hbm, w_hbm, c_hbm, tab_hbm, out_hbm,
                  idx_v, blk_v, x_v, w_v, c_v, sem):
        wid = lax.axis_index("s") * 2 + lax.axis_index("c")
        base = wid * _BPW
        pltpu.sync_copy(w_hbm, w_v)
        pltpu.sync_copy(c_hbm, c_v)

        def chunk_body(g, carry):
            b0 = base + g * _NB
            pltpu.sync_copy(gid_hbm.at[pl.ds(b0, _NB)], idx_v)

            def fire(i, c2):
                pltpu.async_copy(tab_hbm.at[idx_v.at[i]],
                                 blk_v.at[pl.ds(i * _NR + _NCONT, _F)], sem)
                return c2

            lax.fori_loop(0, _NB, fire, 0)
            pltpu.sync_copy(x_hbm.at[pl.ds(b0 * _NCONT, _NB * _NCONT)],
                            x_v.at[pl.ds(0, _NB * _NCONT)])

            def b_body(i, c2):
                xv = x_v[pl.ds(i * _NCONT, 16)]
                for n in range(_NCONT):
                    xs = xv[n]
                    for h in range(2):
                        sl = pl.ds(h * 16, 16)
                        blk_v[i * _NR + n, sl] = w_v[n, sl] * xs + c_v[n, sl]
                return c2

            lax.fori_loop(0, _NB, b_body, 0)

            def drain(i, c2):
                pltpu.make_async_copy(tab_hbm.at[idx_v.at[i]],
                                      blk_v.at[pl.ds(i * _NR + _NCONT, _F)],
                                      sem).wait()
                return c2

            lax.fori_loop(0, _NB, drain, 0)
            pltpu.sync_copy(blk_v,
                            out_hbm.at[pl.ds(b0 * _NR, _NB * _NR)])
            return carry

        lax.fori_loop(0, _NCHUNK, chunk_body, 0)

    return sc_kernel


_SC_KERNEL = _make_sc_kernel()


def kernel(x, categorical, cont_w, cont_b, bn_gamma, bn_beta, bn_mean, bn_var,
           tables):
    eps = 1e-5
    s = bn_gamma / jnp.sqrt(bn_var + eps)
    t = bn_beta - bn_mean * s
    w_fold = cont_w * s[:, None]
    c_fold = cont_w * t[:, None] + cont_b
    tab = tables.reshape(_F * _V, _D)
    gid = categorical + (jnp.arange(_F, dtype=jnp.int32) * _V)[None, :]
    out = _SC_KERNEL(gid, x.reshape(-1), w_fold, c_fold, tab)
    return out.reshape(_B, _NR, _D)


# speedup vs baseline: 2.9030x; 2.9030x over previous
import functools

import jax
import jax.numpy as jnp
from jax import lax
from jax.experimental import pallas as pl
from jax.experimental.pallas import tpu as pltpu
from jax.experimental.pallas import tpu_sc as plsc

_B = 16384
_NCONT = 13
_D = 32
_F = 26
_NR = _NCONT + _F
_V = 100001

_CB = 2048
_NCHUNK = _B // _CB
_VPAD = 100016


def _make_sc_kernel():
    mesh = plsc.VectorSubcoreMesh(core_axis_name="c", subcore_axis_name="s")

    @functools.partial(
        pl.kernel,
        mesh=mesh,
        out_type=jax.ShapeDtypeStruct((_NR, _D, _B), jnp.float32),
        compiler_params=pltpu.CompilerParams(
            use_tc_tiling_on_sc=False, needs_layout_passes=False),
        scratch_types=[
            pltpu.VMEM((_VPAD,), jnp.float32),
            pltpu.VMEM((2, _CB), jnp.int32),
            pltpu.VMEM((2, _CB), jnp.float32),
            pltpu.VMEM((2, _CB), jnp.float32),
            pltpu.VMEM((16,), jnp.float32),
            pltpu.VMEM((16,), jnp.float32),
            pltpu.SemaphoreType.DMA,
            pltpu.SemaphoreType.DMA,
            pltpu.SemaphoreType.DMA,
        ],
    )
    def sc_kernel(cat_hbm, x_hbm, w_hbm, c_hbm, tab_hbm, out_hbm,
                  vec_v, cb_v, xb_v, ob_v, w_v, c_v,
                  sem_v, sem_i, sem_o):
        wid = lax.axis_index("s") * 2 + lax.axis_index("c")

        pltpu.sync_copy(w_hbm.at[wid], w_v)
        pltpu.sync_copy(c_hbm.at[wid], c_v)
        wv = w_v[pl.ds(0, 16)]
        cv = c_v[pl.ds(0, 16)]

        def cont_row(n):
            wn = wv[n]
            cn = cv[n]
            copies = []
            for k in range(_NCHUNK):
                par = k % 2
                pltpu.sync_copy(x_hbm.at[n, pl.ds(k * _CB, _CB)],
                                xb_v.at[par])

                def fma(j, c2):
                    sl = pl.ds(j * 16, 16)
                    ob_v[par, sl] = xb_v[par, sl] * wn + cn
                    return c2

                lax.fori_loop(0, _CB // 16, fma, 0)
                if len(copies) == 2:
                    copies.pop(0).wait()
                copies.append(pltpu.async_copy(
                    ob_v.at[par],
                    out_hbm.at[n, wid, pl.ds(k * _CB, _CB)], sem_o))
            for cp in copies:
                cp.wait()

        for n in range(_NCONT):
            cont_row(n)

        def cat_row(f, _):
            vec_cp = pltpu.async_copy(tab_hbm.at[f, wid],
                                      vec_v.at[pl.ds(0, _V)], sem_v)
            pltpu.sync_copy(cat_hbm.at[f, pl.ds(0, _CB)], cb_v.at[0])
            vec_cp.wait()
            copies = []
            for k in range(_NCHUNK):
                par = k % 2
                if k + 1 < _NCHUNK:
                    nxt = pltpu.async_copy(
                        cat_hbm.at[f, pl.ds((k + 1) * _CB, _CB)],
                        cb_v.at[1 - par], sem_i)

                def gath(j, c2):
                    sl = pl.ds(j * 16, 16)
                    idx = cb_v[par, sl]
                    ob_v[par, sl] = plsc.load_gather(vec_v, [idx])
                    return c2

                lax.fori_loop(0, _CB // 16, gath, 0)
                if len(copies) == 2:
                    copies.pop(0).wait()
                copies.append(pltpu.async_copy(
                    ob_v.at[par],
                    out_hbm.at[_NCONT + f, wid, pl.ds(k * _CB, _CB)],
                    sem_o))
                if k + 1 < _NCHUNK:
                    nxt.wait()
            for cp in copies:
                cp.wait()
            return _

        lax.fori_loop(0, _F, cat_row, 0)

    return sc_kernel


_SC_KERNEL = _make_sc_kernel()


def kernel(x, categorical, cont_w, cont_b, bn_gamma, bn_beta, bn_mean, bn_var,
           tables):
    eps = 1e-5
    s = bn_gamma / jnp.sqrt(bn_var + eps)
    t = bn_beta - bn_mean * s
    w_fold = cont_w * s[:, None]
    c_fold = cont_w * t[:, None] + cont_b
    w_t = jnp.zeros((_D, 16), jnp.float32).at[:, :_NCONT].set(w_fold.T)
    c_t = jnp.zeros((_D, 16), jnp.float32).at[:, :_NCONT].set(c_fold.T)
    tab_t = jnp.transpose(tables, (0, 2, 1))
    cat_t = categorical.T
    x_t = x.T
    out_t = _SC_KERNEL(cat_t, x_t, w_t, c_t, tab_t)
    return jnp.transpose(out_t, (2, 0, 1))


# speedup vs baseline: 10.2161x; 3.5191x over previous
import functools

import jax
import jax.numpy as jnp
from jax import lax
from jax.experimental import pallas as pl
from jax.experimental.pallas import tpu as pltpu
from jax.experimental.pallas import tpu_sc as plsc

_B = 16384
_NCONT = 13
_D = 32
_F = 26
_NR = _NCONT + _F
_V = 100001
_VA = 99968
_VT = _V - _VA
_VROW = 100008
_TBL = _F * _D * _VROW

_CB = 2048
_NCHUNK = _B // _CB
_NSLAB = _F * (_D // 8)


_CHW = 1408
_NCH = _VA // _CHW
_GRP = 4
_NGRP = 18
_AROW = _GRP * _CHW + 40
_ABUF = 8 * _AROW


def _make_reformat_kernel():
    mesh = plsc.VectorSubcoreMesh(core_axis_name="c", subcore_axis_name="s")

    @functools.partial(
        pl.kernel,
        mesh=mesh,
        out_type=jax.ShapeDtypeStruct((_TBL,), jnp.float32),
        compiler_params=pltpu.CompilerParams(use_tc_tiling_on_sc=True),
        scratch_types=[
            pltpu.VMEM((2, 8, _CHW), jnp.float32),
            pltpu.VMEM((2 * _ABUF,), jnp.float32),
            pltpu.VMEM((8, 48), jnp.float32),
            pltpu.SemaphoreType.DMA,
            pltpu.SemaphoreType.DMA,
        ],
    )
    def reformat(tab_hbm, tail_hbm, flat_hbm, chk_v, asm_v, tl_v,
                 sem_r, sem_w):
        wid = lax.axis_index("s") * 2 + lax.axis_index("c")

        def do_slab(s):
            f = s // 4
            d0 = pl.multiple_of((s % 4) * 8, 8)
            base = (f * _D + d0) * _VROW

            def fetch(c):
                return pltpu.async_copy(
                    tab_hbm.at[f, pl.ds(d0, 8), pl.ds(c * _CHW, _CHW)],
                    chk_v.at[c % 2], sem_r)

            tail_cp = pltpu.async_copy(
                tail_hbm.at[f, pl.ds(d0, 8), pl.ds(0, 48)], tl_v, sem_r)
            cur = fetch(0)
            wlists = [[], []]
            for g in range(_NGRP):
                gpar = g % 2
                abase = gpar * _ABUF
                for cp in wlists[gpar]:
                    cp.wait()
                wlists[gpar] = []
                ng = _GRP if g < _NGRP - 1 else _NCH - (_NGRP - 1) * _GRP
                for cc in range(ng):
                    c = g * _GRP + cc
                    cur.wait()
                    if c + 1 < _NCH:
                        nxt = fetch(c + 1)
                    par = c % 2
                    cbase = abase + cc * 11 * 128

                    def dsw(t, carry, par=par, cbase=cbase):
                        k = t % 8
                        j = t // 8
                        src_off = j * 128
                        dst_off = cbase + k * _AROW + j * 128
                        for i in range(8):
                            asm_v[pl.ds(dst_off + i * 16, 16)] = (
                                chk_v[par, k, pl.ds(src_off + i * 16, 16)])
                        return carry

                    lax.fori_loop(0, 88, dsw, 0)
                    if c + 1 < _NCH:
                        cur = nxt
                if g == _NGRP - 1:
                    tail_cp.wait()
                    for k in range(8):
                        for i in range(3):
                            asm_v[pl.ds(abase + k * _AROW + ng * _CHW
                                        + i * 16, 16)] = (
                                tl_v[k, pl.ds(i * 16, 16)])
                width = ng * _CHW + (40 if g == _NGRP - 1 else 0)
                gbase = base + g * _GRP * _CHW
                for k in range(8):
                    wlists[gpar].append(pltpu.async_copy(
                        asm_v.at[pl.ds(abase + k * _AROW, width)],
                        flat_hbm.at[pl.ds(gbase + k * _VROW, width)],
                        sem_w))
            for wl in wlists:
                for cp in wl:
                    cp.wait()

        def slab_loop(m, carry):
            s = wid + 32 * m

            @pl.when(s < _NSLAB)
            def _go():
                do_slab(s)

            return carry

        lax.fori_loop(0, 4, slab_loop, 0)

    return reformat


def _make_lookup_kernel():
    mesh = plsc.VectorSubcoreMesh(core_axis_name="c", subcore_axis_name="s")

    @functools.partial(
        pl.kernel,
        mesh=mesh,
        out_type=jax.ShapeDtypeStruct((_NR, _D, _B), jnp.float32),
        compiler_params=pltpu.CompilerParams(
            use_tc_tiling_on_sc=False, needs_layout_passes=False),
        scratch_types=[
            pltpu.VMEM((_VROW + 8,), jnp.float32),
            pltpu.VMEM((2, _CB), jnp.int32),
            pltpu.VMEM((2, _CB), jnp.float32),
            pltpu.VMEM((2, _CB), jnp.float32),
            pltpu.VMEM((16,), jnp.float32),
            pltpu.VMEM((16,), jnp.float32),
            pltpu.SemaphoreType.DMA,
            pltpu.SemaphoreType.DMA,
            pltpu.SemaphoreType.DMA,
        ],
    )
    def lookup(cat_hbm, x_hbm, w_hbm, c_hbm, flat_hbm, out_hbm,
               vec_v, cb_v, xb_v, ob_v, w_v, c_v,
               sem_v, sem_i, sem_o):
        wid = lax.axis_index("s") * 2 + lax.axis_index("c")

        pltpu.sync_copy(w_hbm.at[wid], w_v)
        pltpu.sync_copy(c_hbm.at[wid], c_v)
        wv = w_v[pl.ds(0, 16)]
        cv = c_v[pl.ds(0, 16)]

        def cont_row(n):
            wn = wv[n]
            cn = cv[n]
            copies = []
            for k in range(_NCHUNK):
                par = k % 2
                pltpu.sync_copy(x_hbm.at[n, pl.ds(k * _CB, _CB)],
                                xb_v.at[par])

                def fma(j, c2):
                    sl = pl.ds(j * 16, 16)
                    ob_v[par, sl] = xb_v[par, sl] * wn + cn
                    return c2

                lax.fori_loop(0, _CB // 16, fma, 0)
                if len(copies) == 2:
                    copies.pop(0).wait()
                copies.append(pltpu.async_copy(
                    ob_v.at[par],
                    out_hbm.at[n, wid, pl.ds(k * _CB, _CB)], sem_o))
            for cp in copies:
                cp.wait()

        for n in range(_NCONT):
            cont_row(n)

        def cat_row(f, _):
            vec_cp = pltpu.async_copy(
                flat_hbm.at[pl.ds((f * _D + wid) * _VROW, _VROW)],
                vec_v.at[pl.ds(0, _VROW)], sem_v)
            pltpu.sync_copy(cat_hbm.at[f, pl.ds(0, _CB)], cb_v.at[0])
            vec_cp.wait()
            copies = []
            for k in range(_NCHUNK):
                par = k % 2
                if k + 1 < _NCHUNK:
                    nxt = pltpu.async_copy(
                        cat_hbm.at[f, pl.ds((k + 1) * _CB, _CB)],
                        cb_v.at[1 - par], sem_i)

                def gath(j, c2):
                    sl = pl.ds(j * 16, 16)
                    idx = cb_v[par, sl]
                    ob_v[par, sl] = plsc.load_gather(vec_v, [idx])
                    return c2

                lax.fori_loop(0, _CB // 16, gath, 0)
                if len(copies) == 2:
                    copies.pop(0).wait()
                copies.append(pltpu.async_copy(
                    ob_v.at[par],
                    out_hbm.at[_NCONT + f, wid, pl.ds(k * _CB, _CB)],
                    sem_o))
                if k + 1 < _NCHUNK:
                    nxt.wait()
            for cp in copies:
                cp.wait()
            return _

        lax.fori_loop(0, _F, cat_row, 0)

    return lookup


_REFORMAT = _make_reformat_kernel()
_LOOKUP = _make_lookup_kernel()


def kernel(x, categorical, cont_w, cont_b, bn_gamma, bn_beta, bn_mean, bn_var,
           tables):
    eps = 1e-5
    s = bn_gamma / jnp.sqrt(bn_var + eps)
    t = bn_beta - bn_mean * s
    w_fold = cont_w * s[:, None]
    c_fold = cont_w * t[:, None] + cont_b
    w_t = jnp.zeros((_D, 16), jnp.float32).at[:, :_NCONT].set(w_fold.T)
    c_t = jnp.zeros((_D, 16), jnp.float32).at[:, :_NCONT].set(c_fold.T)
    tab_t = jnp.transpose(tables, (0, 2, 1))
    cat_t = categorical.T
    x_t = x.T
    tail = jnp.zeros((_F, _D, 48), jnp.float32).at[:, :, :_VT].set(
        jnp.transpose(tables[:, _VA:, :], (0, 2, 1)))
    flat = _REFORMAT(tab_t, tail)
    out_t = _LOOKUP(cat_t, x_t, w_t, c_t, flat)
    return jnp.transpose(out_t, (2, 0, 1))


# speedup vs baseline: 11.3633x; 1.1123x over previous
import functools

import jax
import jax.numpy as jnp
from jax import lax
from jax.experimental import pallas as pl
from jax.experimental.pallas import tpu as pltpu
from jax.experimental.pallas import tpu_sc as plsc

_B = 16384
_NCONT = 13
_D = 32
_F = 26
_NR = _NCONT + _F
_V = 100001
_VA = 99968
_VT = _V - _VA
_VROW = 100008
_TBL = _F * _D * _VROW

_CB = 2048
_NCHUNK = _B // _CB
_NSLAB = _F * (_D // 8)


_CHW = 1408
_NCH = _VA // _CHW
_GRP = 4
_NGRP = 18
_AROW = _GRP * _CHW + 40
_ABUF = 8 * _AROW


def _make_reformat_kernel():
    mesh = plsc.VectorSubcoreMesh(core_axis_name="c", subcore_axis_name="s")

    @functools.partial(
        pl.kernel,
        mesh=mesh,
        out_type=jax.ShapeDtypeStruct((_TBL,), jnp.float32),
        compiler_params=pltpu.CompilerParams(use_tc_tiling_on_sc=True),
        scratch_types=[
            pltpu.VMEM((2, 8, _CHW), jnp.float32),
            pltpu.VMEM((2 * _ABUF,), jnp.float32),
            pltpu.VMEM((8, 48), jnp.float32),
            pltpu.SemaphoreType.DMA,
            pltpu.SemaphoreType.DMA,
            pltpu.SemaphoreType.DMA,
        ],
    )
    def reformat(tab_hbm, tail_hbm, flat_hbm, chk_v, asm_v, tl_v,
                 sem_r, sem_w, sem_t):
        wid = lax.axis_index("s") * 2 + lax.axis_index("c")

        def do_slab(s):
            f = s // 4
            d0 = pl.multiple_of((s % 4) * 8, 8)
            base = (f * _D + d0) * _VROW

            def fetch(c):
                return pltpu.async_copy(
                    tab_hbm.at[f, pl.ds(d0, 8), pl.ds(c * _CHW, _CHW)],
                    chk_v.at[c % 2], sem_r)

            tail_cp = pltpu.async_copy(
                tail_hbm.at[f, pl.ds(d0, 8), pl.ds(0, 48)], tl_v, sem_t)
            fetch(0)
            wlists = [[], []]
            for g in range(_NGRP):
                gpar = g % 2
                abase = gpar * _ABUF
                for cp in wlists[gpar]:
                    cp.wait()
                wlists[gpar] = []
                ng = _GRP if g < _NGRP - 1 else _NCH - (_NGRP - 1) * _GRP

                def chunk_body(cc, carry, g=g):
                    c = g * _GRP + cc

                    @pl.when(c + 1 < _NCH)
                    def _pf():
                        pltpu.async_copy(
                            tab_hbm.at[f, pl.ds(d0, 8),
                                       pl.ds((c + 1) * _CHW, _CHW)],
                            chk_v.at[(c + 1) % 2], sem_r)

                    par = c % 2
                    pltpu.make_async_copy(
                        tab_hbm.at[f, pl.ds(d0, 8), pl.ds(0, _CHW)],
                        chk_v.at[par], sem_r).wait()
                    cbase = abase + cc * 11 * 128

                    def dsw(t, c2):
                        k0 = (t % 4) * 2
                        j = t // 4
                        src_off = j * 128
                        dst_off = cbase + j * 128
                        for kk in range(2):
                            for i in range(8):
                                asm_v[pl.ds(dst_off + (k0 + kk) * _AROW
                                            + i * 16, 16)] = (
                                    chk_v[par, k0 + kk,
                                          pl.ds(src_off + i * 16, 16)])
                        return c2

                    lax.fori_loop(0, 44, dsw, 0)
                    return carry

                lax.fori_loop(0, ng, chunk_body, 0)
                if g == _NGRP - 1:
                    tail_cp.wait()
                    for k in range(8):
                        for i in range(3):
                            asm_v[pl.ds(abase + k * _AROW + ng * _CHW
                                        + i * 16, 16)] = (
                                tl_v[k, pl.ds(i * 16, 16)])
                width = ng * _CHW + (40 if g == _NGRP - 1 else 0)
                gbase = base + g * _GRP * _CHW
                for k in range(8):
                    wlists[gpar].append(pltpu.async_copy(
                        asm_v.at[pl.ds(abase + k * _AROW, width)],
                        flat_hbm.at[pl.ds(gbase + k * _VROW, width)],
                        sem_w))
            for wl in wlists:
                for cp in wl:
                    cp.wait()

        def slab_loop(m, carry):
            s = wid + 32 * m

            @pl.when(s < _NSLAB)
            def _go():
                do_slab(s)

            return carry

        lax.fori_loop(0, 4, slab_loop, 0)

    return reformat


def _make_lookup_kernel():
    mesh = plsc.VectorSubcoreMesh(core_axis_name="c", subcore_axis_name="s")

    @functools.partial(
        pl.kernel,
        mesh=mesh,
        out_type=jax.ShapeDtypeStruct((_NR, _D, _B), jnp.float32),
        compiler_params=pltpu.CompilerParams(
            use_tc_tiling_on_sc=False, needs_layout_passes=False),
        scratch_types=[
            pltpu.VMEM((_VROW + 8,), jnp.float32),
            pltpu.VMEM((2, _CB), jnp.int32),
            pltpu.VMEM((2, _CB), jnp.float32),
            pltpu.VMEM((2, _CB), jnp.float32),
            pltpu.VMEM((16,), jnp.float32),
            pltpu.VMEM((16,), jnp.float32),
            pltpu.SemaphoreType.DMA,
            pltpu.SemaphoreType.DMA,
            pltpu.SemaphoreType.DMA,
        ],
    )
    def lookup(cat_hbm, x_hbm, w_hbm, c_hbm, flat_hbm, out_hbm,
               vec_v, cb_v, xb_v, ob_v, w_v, c_v,
               sem_v, sem_i, sem_o):
        wid = lax.axis_index("s") * 2 + lax.axis_index("c")

        pltpu.sync_copy(w_hbm.at[wid], w_v)
        pltpu.sync_copy(c_hbm.at[wid], c_v)
        wv = w_v[pl.ds(0, 16)]
        cv = c_v[pl.ds(0, 16)]

        xrow = pltpu.async_copy(x_hbm.at[0], vec_v.at[pl.ds(0, _B)], sem_i)

        def cont_row(n):
            wn = wv[n]
            cn = cv[n]
            xoff = (n % 2) * _B
            copies = []
            for k in range(_NCHUNK):
                par = k % 2

                def fma(j, c2):
                    for u in range(4):
                        off = j * 64 + u * 16
                        ob_v[par, pl.ds(off, 16)] = (
                            vec_v[pl.ds(xoff + k * _CB + off, 16)] * wn + cn)
                    return c2

                lax.fori_loop(0, _CB // 64, fma, 0)
                if len(copies) == 2:
                    copies.pop(0).wait()
                copies.append(pltpu.async_copy(
                    ob_v.at[par],
                    out_hbm.at[n, wid, pl.ds(k * _CB, _CB)], sem_o))
            for cp in copies:
                cp.wait()

        for n in range(_NCONT):
            xrow.wait()
            if n + 1 < _NCONT:
                xrow = pltpu.async_copy(
                    x_hbm.at[n + 1],
                    vec_v.at[pl.ds(((n + 1) % 2) * _B, _B)], sem_i)
            cont_row(n)

        def cat_row(f, _):
            vec_cp = pltpu.async_copy(
                flat_hbm.at[pl.ds((f * _D + wid) * _VROW, _VROW)],
                vec_v.at[pl.ds(0, _VROW)], sem_v)
            pltpu.sync_copy(cat_hbm.at[f, pl.ds(0, _CB)], cb_v.at[0])
            vec_cp.wait()
            copies = []
            for k in range(_NCHUNK):
                par = k % 2
                if k + 1 < _NCHUNK:
                    nxt = pltpu.async_copy(
                        cat_hbm.at[f, pl.ds((k + 1) * _CB, _CB)],
                        cb_v.at[1 - par], sem_i)

                def gath(j, c2):
                    for u in range(4):
                        sl = pl.ds(j * 64 + u * 16, 16)
                        idx = cb_v[par, sl]
                        ob_v[par, sl] = plsc.load_gather(vec_v, [idx])
                    return c2

                lax.fori_loop(0, _CB // 64, gath, 0)
                if len(copies) == 2:
                    copies.pop(0).wait()
                copies.append(pltpu.async_copy(
                    ob_v.at[par],
                    out_hbm.at[_NCONT + f, wid, pl.ds(k * _CB, _CB)],
                    sem_o))
                if k + 1 < _NCHUNK:
                    nxt.wait()
            for cp in copies:
                cp.wait()
            return _

        lax.fori_loop(0, _F, cat_row, 0)

    return lookup


_REFORMAT = _make_reformat_kernel()
_LOOKUP = _make_lookup_kernel()


def kernel(x, categorical, cont_w, cont_b, bn_gamma, bn_beta, bn_mean, bn_var,
           tables):
    eps = 1e-5
    s = bn_gamma / jnp.sqrt(bn_var + eps)
    t = bn_beta - bn_mean * s
    w_fold = cont_w * s[:, None]
    c_fold = cont_w * t[:, None] + cont_b
    w_t = jnp.zeros((_D, 16), jnp.float32).at[:, :_NCONT].set(w_fold.T)
    c_t = jnp.zeros((_D, 16), jnp.float32).at[:, :_NCONT].set(c_fold.T)
    tab_t = jnp.transpose(tables, (0, 2, 1))
    cat_t = categorical.T
    x_t = x.T
    tail = jnp.zeros((_F, _D, 48), jnp.float32).at[:, :, :_VT].set(
        jnp.transpose(tables[:, _VA:, :], (0, 2, 1)))
    flat = _REFORMAT(tab_t, tail)
    out_t = _LOOKUP(cat_t, x_t, w_t, c_t, flat)
    return jnp.transpose(out_t, (2, 0, 1))


# speedup vs baseline: 12.1647x; 1.0705x over previous
import functools

import jax
import jax.numpy as jnp
from jax import lax
from jax.experimental import pallas as pl
from jax.experimental.pallas import tpu as pltpu
from jax.experimental.pallas import tpu_sc as plsc

_B = 16384
_NCONT = 13
_D = 32
_F = 26
_NR = _NCONT + _F
_V = 100001
_VA = 99968
_VT = _V - _VA
_VROW = 100016
_TBL = _F * _D * _VROW

_CB = 2048
_NCHUNK = _B // _CB
_NSLAB = _F * (_D // 8)


_CHW = 1408
_NCH = _VA // _CHW
_GRP = 4
_NGRP = 18
_AROW = _GRP * _CHW + 40
_ABUF = 8 * _AROW


def _make_reformat_kernel():
    mesh = plsc.VectorSubcoreMesh(core_axis_name="c", subcore_axis_name="s")

    @functools.partial(
        pl.kernel,
        mesh=mesh,
        out_type=jax.ShapeDtypeStruct((_TBL,), jnp.float32),
        compiler_params=pltpu.CompilerParams(use_tc_tiling_on_sc=True),
        scratch_types=[
            pltpu.VMEM((2, 8, _CHW), jnp.float32),
            pltpu.VMEM((2 * _ABUF,), jnp.float32),
            pltpu.VMEM((8, 48), jnp.float32),
            pltpu.SemaphoreType.DMA,
            pltpu.SemaphoreType.DMA,
            pltpu.SemaphoreType.DMA,
        ],
    )
    def reformat(tab_hbm, tail_hbm, flat_hbm, chk_v, asm_v, tl_v,
                 sem_r, sem_w, sem_t):
        wid = lax.axis_index("s") * 2 + lax.axis_index("c")

        def do_slab(s, g_lo, g_hi):
            f = s // 4
            d0 = pl.multiple_of((s % 4) * 8, 8)
            base = (f * _D + d0) * _VROW
            end_chunk = _NCH if g_hi == _NGRP else g_hi * _GRP

            def fetch(c):
                return pltpu.async_copy(
                    tab_hbm.at[f, pl.ds(d0, 8), pl.ds(c * _CHW, _CHW)],
                    chk_v.at[c % 2], sem_r)

            if g_hi == _NGRP:
                tail_cp = pltpu.async_copy(
                    tail_hbm.at[f, pl.ds(d0, 8), pl.ds(0, 48)], tl_v, sem_t)
            fetch(g_lo * _GRP)
            wlists = [[], []]
            for g in range(g_lo, g_hi):
                gpar = g % 2
                abase = gpar * _ABUF
                for cp in wlists[gpar]:
                    cp.wait()
                wlists[gpar] = []
                ng = _GRP if g < _NGRP - 1 else _NCH - (_NGRP - 1) * _GRP

                def chunk_body(cc, carry, g=g):
                    c = g * _GRP + cc

                    @pl.when(c + 1 < end_chunk)
                    def _pf():
                        pltpu.async_copy(
                            tab_hbm.at[f, pl.ds(d0, 8),
                                       pl.ds((c + 1) * _CHW, _CHW)],
                            chk_v.at[(c + 1) % 2], sem_r)

                    par = c % 2
                    pltpu.make_async_copy(
                        tab_hbm.at[f, pl.ds(d0, 8), pl.ds(0, _CHW)],
                        chk_v.at[par], sem_r).wait()
                    cbase = abase + cc * 11 * 128

                    def dsw(t, c2):
                        k0 = (t % 4) * 2
                        j = t // 4
                        src_off = j * 128
                        dst_off = cbase + j * 128
                        for kk in range(2):
                            for i in range(8):
                                asm_v[pl.ds(dst_off + (k0 + kk) * _AROW
                                            + i * 16, 16)] = (
                                    chk_v[par, k0 + kk,
                                          pl.ds(src_off + i * 16, 16)])
                        return c2

                    lax.fori_loop(0, 44, dsw, 0)
                    return carry

                lax.fori_loop(0, ng, chunk_body, 0)
                if g == _NGRP - 1:
                    tail_cp.wait()
                    for k in range(8):
                        for i in range(3):
                            asm_v[pl.ds(abase + k * _AROW + ng * _CHW
                                        + i * 16, 16)] = (
                                tl_v[k, pl.ds(i * 16, 16)])
                width = ng * _CHW + (40 if g == _NGRP - 1 else 0)
                gbase = base + g * _GRP * _CHW
                for k in range(8):
                    wlists[gpar].append(pltpu.async_copy(
                        asm_v.at[pl.ds(abase + k * _AROW, width)],
                        flat_hbm.at[pl.ds(gbase + k * _VROW, width)],
                        sem_w))
            for wl in wlists:
                for cp in wl:
                    cp.wait()

        def slab_loop(m, carry):
            item = wid + 32 * m

            @pl.when(item < 2 * _NSLAB)
            def _go():
                s = item // 2

                @pl.when(item % 2 == 0)
                def _lo():
                    do_slab(s, 0, _NGRP // 2)

                @pl.when(item % 2 == 1)
                def _hi():
                    do_slab(s, _NGRP // 2, _NGRP)

            return carry

        lax.fori_loop(0, 7, slab_loop, 0)

    return reformat


def _make_lookup_kernel():
    mesh = plsc.VectorSubcoreMesh(core_axis_name="c", subcore_axis_name="s")

    @functools.partial(
        pl.kernel,
        mesh=mesh,
        out_type=jax.ShapeDtypeStruct((_NR, _D, _B), jnp.float32),
        compiler_params=pltpu.CompilerParams(
            use_tc_tiling_on_sc=False, needs_layout_passes=False),
        scratch_types=[
            pltpu.VMEM((_VROW + 8,), jnp.float32),
            pltpu.VMEM((2, _CB), jnp.int32),
            pltpu.VMEM((2, _CB), jnp.float32),
            pltpu.VMEM((2, _CB), jnp.float32),
            pltpu.VMEM((16,), jnp.float32),
            pltpu.VMEM((16,), jnp.float32),
            pltpu.SemaphoreType.DMA,
            pltpu.SemaphoreType.DMA,
            pltpu.SemaphoreType.DMA,
        ],
    )
    def lookup(cat_hbm, x_hbm, w_hbm, c_hbm, flat_hbm, out_hbm,
               vec_v, cb_v, xb_v, ob_v, w_v, c_v,
               sem_v, sem_i, sem_o):
        wid = lax.axis_index("s") * 2 + lax.axis_index("c")

        pltpu.sync_copy(w_hbm.at[wid], w_v)
        pltpu.sync_copy(c_hbm.at[wid], c_v)
        wv = w_v[pl.ds(0, 16)]
        cv = c_v[pl.ds(0, 16)]

        xrow = pltpu.async_copy(x_hbm.at[0], vec_v.at[pl.ds(0, _B)], sem_i)

        def cont_row(n):
            wn = wv[n]
            cn = cv[n]
            xoff = (n % 2) * _B
            copies = []
            for k in range(_NCHUNK):
                par = k % 2

                def fma(j, c2):
                    for u in range(4):
                        off = j * 64 + u * 16
                        ob_v[par, pl.ds(off, 16)] = (
                            vec_v[pl.ds(xoff + k * _CB + off, 16)] * wn + cn)
                    return c2

                lax.fori_loop(0, _CB // 64, fma, 0)
                if len(copies) == 2:
                    copies.pop(0).wait()
                copies.append(pltpu.async_copy(
                    ob_v.at[par],
                    out_hbm.at[n, wid, pl.ds(k * _CB, _CB)], sem_o))
            for cp in copies:
                cp.wait()

        for n in range(_NCONT):
            xrow.wait()
            if n + 1 < _NCONT:
                xrow = pltpu.async_copy(
                    x_hbm.at[n + 1],
                    vec_v.at[pl.ds(((n + 1) % 2) * _B, _B)], sem_i)
            cont_row(n)

        def cat_row(f, _):
            vec_cp = pltpu.async_copy(
                flat_hbm.at[pl.ds((f * _D + wid) * _VROW, _VROW)],
                vec_v.at[pl.ds(0, _VROW)], sem_v)
            pltpu.sync_copy(cat_hbm.at[f, pl.ds(0, _CB)], cb_v.at[0])
            vec_cp.wait()
            copies = []
            for k in range(_NCHUNK):
                par = k % 2
                if k + 1 < _NCHUNK:
                    nxt = pltpu.async_copy(
                        cat_hbm.at[f, pl.ds((k + 1) * _CB, _CB)],
                        cb_v.at[1 - par], sem_i)

                def gath(j, c2):
                    for u in range(4):
                        sl = pl.ds(j * 64 + u * 16, 16)
                        idx = cb_v[par, sl]
                        ob_v[par, sl] = plsc.load_gather(vec_v, [idx])
                    return c2

                lax.fori_loop(0, _CB // 64, gath, 0)
                if len(copies) == 2:
                    copies.pop(0).wait()
                copies.append(pltpu.async_copy(
                    ob_v.at[par],
                    out_hbm.at[_NCONT + f, wid, pl.ds(k * _CB, _CB)],
                    sem_o))
                if k + 1 < _NCHUNK:
                    nxt.wait()
            for cp in copies:
                cp.wait()
            return _

        lax.fori_loop(0, _F, cat_row, 0)

    return lookup


_REFORMAT = _make_reformat_kernel()
_LOOKUP = _make_lookup_kernel()


def kernel(x, categorical, cont_w, cont_b, bn_gamma, bn_beta, bn_mean, bn_var,
           tables):
    eps = 1e-5
    s = bn_gamma / jnp.sqrt(bn_var + eps)
    t = bn_beta - bn_mean * s
    w_fold = cont_w * s[:, None]
    c_fold = cont_w * t[:, None] + cont_b
    w_t = jnp.zeros((_D, 16), jnp.float32).at[:, :_NCONT].set(w_fold.T)
    c_t = jnp.zeros((_D, 16), jnp.float32).at[:, :_NCONT].set(c_fold.T)
    tab_t = jnp.transpose(tables, (0, 2, 1))
    cat_t = categorical.T
    x_t = x.T
    tail = jnp.zeros((_F, _D, 48), jnp.float32).at[:, :, :_VT].set(
        jnp.transpose(tables[:, _VA:, :], (0, 2, 1)))
    flat = _REFORMAT(tab_t, tail)
    out_t = _LOOKUP(cat_t, x_t, w_t, c_t, flat)
    return jnp.transpose(out_t, (2, 0, 1))


# speedup vs baseline: 12.2156x; 1.0042x over previous
import functools

import jax
import jax.numpy as jnp
from jax import lax
from jax.experimental import pallas as pl
from jax.experimental.pallas import tpu as pltpu
from jax.experimental.pallas import tpu_sc as plsc

_B = 16384
_NCONT = 13
_D = 32
_F = 26
_NR = _NCONT + _F
_V = 100001
_VA = 99968
_VT = _V - _VA
_VROW = 100016
_TBL = _F * _D * _VROW

_CB = 2048
_NCHUNK = _B // _CB
_NSLAB = _F * (_D // 8)


_CHW = 1408
_NCH = _VA // _CHW
_GRP = 4
_NGRP = 18
_AROW = _GRP * _CHW + 40
_ABUF = 8 * _AROW


def _make_reformat_kernel():
    mesh = plsc.VectorSubcoreMesh(core_axis_name="c", subcore_axis_name="s")

    @functools.partial(
        pl.kernel,
        mesh=mesh,
        out_type=jax.ShapeDtypeStruct((_TBL,), jnp.float32),
        compiler_params=pltpu.CompilerParams(use_tc_tiling_on_sc=True),
        scratch_types=[
            pltpu.VMEM((2, 8, _CHW), jnp.float32),
            pltpu.VMEM((2 * _ABUF,), jnp.float32),
            pltpu.VMEM((8, 48), jnp.float32),
            pltpu.SemaphoreType.DMA,
            pltpu.SemaphoreType.DMA,
            pltpu.SemaphoreType.DMA,
        ],
    )
    def reformat(tab_hbm, tail_hbm, flat_hbm, chk_v, asm_v, tl_v,
                 sem_r, sem_w, sem_t):
        wid = lax.axis_index("s") * 2 + lax.axis_index("c")

        def do_slab(s, g_lo, g_hi):
            f = s // 4
            d0 = pl.multiple_of((s % 4) * 8, 8)
            base = (f * _D + d0) * _VROW
            end_chunk = _NCH if g_hi == _NGRP else g_hi * _GRP

            def fetch(c):
                return pltpu.async_copy(
                    tab_hbm.at[f, pl.ds(d0, 8), pl.ds(c * _CHW, _CHW)],
                    chk_v.at[c % 2], sem_r)

            if g_hi == _NGRP:
                tail_cp = pltpu.async_copy(
                    tail_hbm.at[f, pl.ds(d0, 8), pl.ds(0, 48)], tl_v, sem_t)
            fetch(g_lo * _GRP)
            wlists = [[], []]
            for g in range(g_lo, g_hi):
                gpar = g % 2
                abase = gpar * _ABUF
                for cp in wlists[gpar]:
                    cp.wait()
                wlists[gpar] = []
                ng = _GRP if g < _NGRP - 1 else _NCH - (_NGRP - 1) * _GRP

                def chunk_body(cc, carry, g=g):
                    c = g * _GRP + cc

                    @pl.when(c + 1 < end_chunk)
                    def _pf():
                        pltpu.async_copy(
                            tab_hbm.at[f, pl.ds(d0, 8),
                                       pl.ds((c + 1) * _CHW, _CHW)],
                            chk_v.at[(c + 1) % 2], sem_r)

                    par = c % 2
                    pltpu.make_async_copy(
                        tab_hbm.at[f, pl.ds(d0, 8), pl.ds(0, _CHW)],
                        chk_v.at[par], sem_r).wait()
                    cbase = abase + cc * 11 * 128

                    def dsw(t, c2):
                        k0 = (t % 2) * 4
                        j = t // 2
                        src_off = j * 128
                        dst_off = cbase + j * 128
                        for kk in range(4):
                            for i in range(8):
                                asm_v[pl.ds(dst_off + (k0 + kk) * _AROW
                                            + i * 16, 16)] = (
                                    chk_v[par, k0 + kk,
                                          pl.ds(src_off + i * 16, 16)])
                        return c2

                    lax.fori_loop(0, 22, dsw, 0)
                    return carry

                lax.fori_loop(0, ng, chunk_body, 0)
                if g == _NGRP - 1:
                    tail_cp.wait()
                    for k in range(8):
                        for i in range(3):
                            asm_v[pl.ds(abase + k * _AROW + ng * _CHW
                                        + i * 16, 16)] = (
                                tl_v[k, pl.ds(i * 16, 16)])
                width = ng * _CHW + (40 if g == _NGRP - 1 else 0)
                gbase = base + g * _GRP * _CHW
                for k in range(8):
                    wlists[gpar].append(pltpu.async_copy(
                        asm_v.at[pl.ds(abase + k * _AROW, width)],
                        flat_hbm.at[pl.ds(gbase + k * _VROW, width)],
                        sem_w))
            for wl in wlists:
                for cp in wl:
                    cp.wait()

        def slab_loop(m, carry):
            item = wid + 32 * m

            @pl.when(item < 2 * _NSLAB)
            def _go():
                s = item // 2

                @pl.when(item % 2 == 0)
                def _lo():
                    do_slab(s, 0, _NGRP // 2)

                @pl.when(item % 2 == 1)
                def _hi():
                    do_slab(s, _NGRP // 2, _NGRP)

            return carry

        lax.fori_loop(0, 7, slab_loop, 0)

    return reformat


def _make_lookup_kernel():
    mesh = plsc.VectorSubcoreMesh(core_axis_name="c", subcore_axis_name="s")

    @functools.partial(
        pl.kernel,
        mesh=mesh,
        out_type=jax.ShapeDtypeStruct((_NR, _D, _B), jnp.float32),
        compiler_params=pltpu.CompilerParams(
            use_tc_tiling_on_sc=False, needs_layout_passes=False),
        scratch_types=[
            pltpu.VMEM((_VROW + 8,), jnp.float32),
            pltpu.VMEM((2, _CB), jnp.int32),
            pltpu.VMEM((2, _CB), jnp.float32),
            pltpu.VMEM((2, _CB), jnp.float32),
            pltpu.VMEM((16,), jnp.float32),
            pltpu.VMEM((16,), jnp.float32),
            pltpu.SemaphoreType.DMA,
            pltpu.SemaphoreType.DMA,
            pltpu.SemaphoreType.DMA,
        ],
    )
    def lookup(cat_hbm, x_hbm, w_hbm, c_hbm, flat_hbm, out_hbm,
               vec_v, cb_v, xb_v, ob_v, w_v, c_v,
               sem_v, sem_i, sem_o):
        wid = lax.axis_index("s") * 2 + lax.axis_index("c")

        pltpu.sync_copy(w_hbm.at[wid], w_v)
        pltpu.sync_copy(c_hbm.at[wid], c_v)
        wv = w_v[pl.ds(0, 16)]
        cv = c_v[pl.ds(0, 16)]

        xrow = pltpu.async_copy(x_hbm.at[0], vec_v.at[pl.ds(0, _B)], sem_i)

        def cont_row(n):
            wn = wv[n]
            cn = cv[n]
            xoff = (n % 2) * _B
            copies = []
            for k in range(_NCHUNK):
                par = k % 2

                def fma(j, c2):
                    for u in range(4):
                        off = j * 64 + u * 16
                        ob_v[par, pl.ds(off, 16)] = (
                            vec_v[pl.ds(xoff + k * _CB + off, 16)] * wn + cn)
                    return c2

                lax.fori_loop(0, _CB // 64, fma, 0)
                if len(copies) == 2:
                    copies.pop(0).wait()
                copies.append(pltpu.async_copy(
                    ob_v.at[par],
                    out_hbm.at[n, wid, pl.ds(k * _CB, _CB)], sem_o))
            for cp in copies:
                cp.wait()

        for n in range(_NCONT):
            xrow.wait()
            if n + 1 < _NCONT:
                xrow = pltpu.async_copy(
                    x_hbm.at[n + 1],
                    vec_v.at[pl.ds(((n + 1) % 2) * _B, _B)], sem_i)
            cont_row(n)

        def cat_row(f, _):
            vec_cp = pltpu.async_copy(
                flat_hbm.at[pl.ds((f * _D + wid) * _VROW, _VROW)],
                vec_v.at[pl.ds(0, _VROW)], sem_v)
            pltpu.sync_copy(cat_hbm.at[f, pl.ds(0, _CB)], cb_v.at[0])
            vec_cp.wait()
            copies = []
            for k in range(_NCHUNK):
                par = k % 2
                if k + 1 < _NCHUNK:
                    nxt = pltpu.async_copy(
                        cat_hbm.at[f, pl.ds((k + 1) * _CB, _CB)],
                        cb_v.at[1 - par], sem_i)

                def gath(j, c2):
                    for u in range(8):
                        sl = pl.ds(j * 128 + u * 16, 16)
                        idx = cb_v[par, sl]
                        ob_v[par, sl] = plsc.load_gather(vec_v, [idx])
                    return c2

                lax.fori_loop(0, _CB // 128, gath, 0)
                if len(copies) == 2:
                    copies.pop(0).wait()
                copies.append(pltpu.async_copy(
                    ob_v.at[par],
                    out_hbm.at[_NCONT + f, wid, pl.ds(k * _CB, _CB)],
                    sem_o))
                if k + 1 < _NCHUNK:
                    nxt.wait()
            for cp in copies:
                cp.wait()
            return _

        lax.fori_loop(0, _F, cat_row, 0)

    return lookup


_REFORMAT = _make_reformat_kernel()
_LOOKUP = _make_lookup_kernel()


def kernel(x, categorical, cont_w, cont_b, bn_gamma, bn_beta, bn_mean, bn_var,
           tables):
    eps = 1e-5
    s = bn_gamma / jnp.sqrt(bn_var + eps)
    t = bn_beta - bn_mean * s
    w_fold = cont_w * s[:, None]
    c_fold = cont_w * t[:, None] + cont_b
    w_t = jnp.zeros((_D, 16), jnp.float32).at[:, :_NCONT].set(w_fold.T)
    c_t = jnp.zeros((_D, 16), jnp.float32).at[:, :_NCONT].set(c_fold.T)
    tab_t = jnp.transpose(tables, (0, 2, 1))
    cat_t = categorical.T
    x_t = x.T
    tail = jnp.zeros((_F, _D, 48), jnp.float32).at[:, :, :_VT].set(
        jnp.transpose(tables[:, _VA:, :], (0, 2, 1)))
    flat = _REFORMAT(tab_t, tail)
    out_t = _LOOKUP(cat_t, x_t, w_t, c_t, flat)
    return jnp.transpose(out_t, (2, 0, 1))


# speedup vs baseline: 12.2635x; 1.0039x over previous
import functools

import jax
import jax.numpy as jnp
from jax import lax
from jax.experimental import pallas as pl
from jax.experimental.pallas import tpu as pltpu
from jax.experimental.pallas import tpu_sc as plsc

_B = 16384
_NCONT = 13
_D = 32
_F = 26
_NR = _NCONT + _F
_V = 100001
_VA = 99968
_VT = _V - _VA
_VROW = 100016
_TBL = _F * _D * _VROW

_CB = 2048
_NCHUNK = _B // _CB
_NSLAB = _F * (_D // 8)


_CHW = 1408
_NCH = _VA // _CHW
_GRP = 4
_NGRP = 18
_AROW = _GRP * _CHW + 40
_ABUF = 8 * _AROW


def _make_reformat_kernel():
    mesh = plsc.VectorSubcoreMesh(core_axis_name="c", subcore_axis_name="s")

    @functools.partial(
        pl.kernel,
        mesh=mesh,
        out_type=jax.ShapeDtypeStruct((_TBL,), jnp.float32),
        compiler_params=pltpu.CompilerParams(use_tc_tiling_on_sc=True),
        scratch_types=[
            pltpu.VMEM((2, 8, _CHW), jnp.float32),
            pltpu.VMEM((2 * _ABUF,), jnp.float32),
            pltpu.VMEM((8, 48), jnp.float32),
            pltpu.SemaphoreType.DMA,
            pltpu.SemaphoreType.DMA,
            pltpu.SemaphoreType.DMA,
        ],
    )
    def reformat(tab_hbm, tail_hbm, flat_hbm, chk_v, asm_v, tl_v,
                 sem_r, sem_w, sem_t):
        wid = lax.axis_index("s") * 2 + lax.axis_index("c")

        def do_slab(s, g_lo, g_hi):
            f = s // 4
            d0 = pl.multiple_of((s % 4) * 8, 8)
            base = (f * _D + d0) * _VROW
            end_chunk = _NCH if g_hi == _NGRP else g_hi * _GRP

            def fetch(c):
                return pltpu.async_copy(
                    tab_hbm.at[f, pl.ds(d0, 8), pl.ds(c * _CHW, _CHW)],
                    chk_v.at[c % 2], sem_r)

            if g_hi == _NGRP:
                tail_cp = pltpu.async_copy(
                    tail_hbm.at[f, pl.ds(d0, 8), pl.ds(0, 48)], tl_v, sem_t)
            fetch(g_lo * _GRP)
            wlists = [[], []]
            for g in range(g_lo, g_hi):
                gpar = g % 2
                abase = gpar * _ABUF
                for cp in wlists[gpar]:
                    cp.wait()
                wlists[gpar] = []
                ng = _GRP if g < _NGRP - 1 else _NCH - (_NGRP - 1) * _GRP

                def chunk_body(cc, carry, g=g):
                    c = g * _GRP + cc

                    @pl.when(c + 1 < end_chunk)
                    def _pf():
                        pltpu.async_copy(
                            tab_hbm.at[f, pl.ds(d0, 8),
                                       pl.ds((c + 1) * _CHW, _CHW)],
                            chk_v.at[(c + 1) % 2], sem_r)

                    par = c % 2
                    pltpu.make_async_copy(
                        tab_hbm.at[f, pl.ds(d0, 8), pl.ds(0, _CHW)],
                        chk_v.at[par], sem_r).wait()
                    cbase = abase + cc * 11 * 128

                    def dsw(t, c2):
                        k0 = (t % 2) * 4
                        j = t // 2
                        src_off = pl.multiple_of(j * 128, 128)
                        dst_off = cbase + j * 128
                        for kk in range(4):
                            for i in range(8):
                                asm_v[pl.ds(dst_off + (k0 + kk) * _AROW
                                            + i * 16, 16)] = (
                                    chk_v[par, k0 + kk,
                                          pl.ds(src_off + i * 16, 16)])
                        return c2

                    lax.fori_loop(0, 22, dsw, 0)
                    return carry

                lax.fori_loop(0, ng, chunk_body, 0)
                if g == _NGRP - 1:
                    tail_cp.wait()
                    for k in range(8):
                        for i in range(3):
                            asm_v[pl.ds(abase + k * _AROW + ng * _CHW
                                        + i * 16, 16)] = (
                                tl_v[k, pl.ds(i * 16, 16)])
                width = ng * _CHW + (40 if g == _NGRP - 1 else 0)
                gbase = base + g * _GRP * _CHW
                for k in range(8):
                    wlists[gpar].append(pltpu.async_copy(
                        asm_v.at[pl.ds(abase + k * _AROW, width)],
                        flat_hbm.at[pl.ds(gbase + k * _VROW, width)],
                        sem_w))
            for wl in wlists:
                for cp in wl:
                    cp.wait()

        def slab_loop(m, carry):
            item = wid + 32 * m

            @pl.when(item < 2 * _NSLAB)
            def _go():
                s = item // 2

                @pl.when(item % 2 == 0)
                def _lo():
                    do_slab(s, 0, _NGRP // 2)

                @pl.when(item % 2 == 1)
                def _hi():
                    do_slab(s, _NGRP // 2, _NGRP)

            return carry

        lax.fori_loop(0, 7, slab_loop, 0)

    return reformat


def _make_lookup_kernel():
    mesh = plsc.VectorSubcoreMesh(core_axis_name="c", subcore_axis_name="s")

    @functools.partial(
        pl.kernel,
        mesh=mesh,
        out_type=jax.ShapeDtypeStruct((_NR, _D, _B), jnp.float32),
        compiler_params=pltpu.CompilerParams(
            use_tc_tiling_on_sc=False, needs_layout_passes=False),
        scratch_types=[
            pltpu.VMEM((_VROW + 8,), jnp.float32),
            pltpu.VMEM((2, _CB), jnp.int32),
            pltpu.VMEM((2, _CB), jnp.float32),
            pltpu.VMEM((2, _CB), jnp.float32),
            pltpu.VMEM((16,), jnp.float32),
            pltpu.VMEM((16,), jnp.float32),
            pltpu.SemaphoreType.DMA,
            pltpu.SemaphoreType.DMA,
            pltpu.SemaphoreType.DMA,
        ],
    )
    def lookup(cat_hbm, x_hbm, w_hbm, c_hbm, flat_hbm, out_hbm,
               vec_v, cb_v, xb_v, ob_v, w_v, c_v,
               sem_v, sem_i, sem_o):
        wid = lax.axis_index("s") * 2 + lax.axis_index("c")

        pltpu.sync_copy(w_hbm.at[wid], w_v)
        pltpu.sync_copy(c_hbm.at[wid], c_v)
        wv = w_v[pl.ds(0, 16)]
        cv = c_v[pl.ds(0, 16)]

        xrow = pltpu.async_copy(x_hbm.at[0], vec_v.at[pl.ds(0, _B)], sem_i)

        def cont_row(n):
            wn = wv[n]
            cn = cv[n]
            xoff = (n % 2) * _B
            copies = []
            for k in range(_NCHUNK):
                par = k % 2

                def fma(j, c2):
                    for u in range(4):
                        off = j * 64 + u * 16
                        ob_v[par, pl.ds(off, 16)] = (
                            vec_v[pl.ds(xoff + k * _CB + off, 16)] * wn + cn)
                    return c2

                lax.fori_loop(0, _CB // 64, fma, 0)
                if len(copies) == 2:
                    copies.pop(0).wait()
                copies.append(pltpu.async_copy(
                    ob_v.at[par],
                    out_hbm.at[n, wid, pl.ds(k * _CB, _CB)], sem_o))
            for cp in copies:
                cp.wait()

        for n in range(_NCONT):
            xrow.wait()
            if n + 1 < _NCONT:
                xrow = pltpu.async_copy(
                    x_hbm.at[n + 1],
                    vec_v.at[pl.ds(((n + 1) % 2) * _B, _B)], sem_i)
            cont_row(n)

        def cat_row(f, _):
            vec_cp = pltpu.async_copy(
                flat_hbm.at[pl.ds((f * _D + wid) * _VROW, _VROW)],
                vec_v.at[pl.ds(0, _VROW)], sem_v)
            pltpu.sync_copy(cat_hbm.at[f, pl.ds(0, _CB)], cb_v.at[0])
            vec_cp.wait()
            copies = []
            for k in range(_NCHUNK):
                par = k % 2
                if k + 1 < _NCHUNK:
                    nxt = pltpu.async_copy(
                        cat_hbm.at[f, pl.ds((k + 1) * _CB, _CB)],
                        cb_v.at[1 - par], sem_i)

                def gath(j, c2):
                    for u in range(8):
                        sl = pl.ds(j * 128 + u * 16, 16)
                        idx = cb_v[par, sl]
                        ob_v[par, sl] = plsc.load_gather(vec_v, [idx])
                    return c2

                lax.fori_loop(0, _CB // 128, gath, 0)
                if len(copies) == 2:
                    copies.pop(0).wait()
                copies.append(pltpu.async_copy(
                    ob_v.at[par],
                    out_hbm.at[_NCONT + f, wid, pl.ds(k * _CB, _CB)],
                    sem_o))
                if k + 1 < _NCHUNK:
                    nxt.wait()
            for cp in copies:
                cp.wait()
            return _

        lax.fori_loop(0, _F, cat_row, 0)

    return lookup


_REFORMAT = _make_reformat_kernel()
_LOOKUP = _make_lookup_kernel()


def kernel(x, categorical, cont_w, cont_b, bn_gamma, bn_beta, bn_mean, bn_var,
           tables):
    eps = 1e-5
    s = bn_gamma / jnp.sqrt(bn_var + eps)
    t = bn_beta - bn_mean * s
    w_fold = cont_w * s[:, None]
    c_fold = cont_w * t[:, None] + cont_b
    w_t = jnp.zeros((_D, 16), jnp.float32).at[:, :_NCONT].set(w_fold.T)
    c_t = jnp.zeros((_D, 16), jnp.float32).at[:, :_NCONT].set(c_fold.T)
    tab_t = jnp.transpose(tables, (0, 2, 1))
    cat_t = categorical.T
    x_t = x.T
    tail = jnp.zeros((_F, _D, 48), jnp.float32).at[:, :, :_VT].set(
        jnp.transpose(tables[:, _VA:, :], (0, 2, 1)))
    flat = _REFORMAT(tab_t, tail)
    out_t = _LOOKUP(cat_t, x_t, w_t, c_t, flat)
    return jnp.transpose(out_t, (2, 0, 1))
